# baseline (speedup 1.0000x reference)
import functools

import jax
import jax.numpy as jnp
from jax.experimental import pallas as pl
from jax.experimental.pallas import tpu as pltpu

N, D, E, TOPK, FF = 2048, 768, 8, 2, 1536
BN = 256
BLK = 128
MAXS = 4096 + E * BLK
NBLK = MAXS // BLK
EPAD = 128


def _attn_router_kernel(x_ref, noise_ref, aw_ref, ab_ref, rw_ref,
                        h_ref, ti_ref, tw_ref):
    x = x_ref[...]
    attn = jax.lax.dot_general(x, aw_ref[...], (((1,), (0,)), ((), ())),
                               preferred_element_type=jnp.float32)
    h = x + (attn + ab_ref[...] + noise_ref[...])
    h_ref[...] = h
    logits = jax.lax.dot_general(h, rw_ref[...], (((1,), (0,)), ((), ())),
                                 preferred_element_type=jnp.float32)
    col = jax.lax.broadcasted_iota(jnp.int32, (BN, EPAD), 1)
    neg = jnp.float32(-1e30)
    masked = jnp.where(col < E, logits, neg)
    m1 = jnp.max(masked, axis=1, keepdims=True)
    i1 = jnp.min(jnp.where(masked == m1, col, EPAD), axis=1, keepdims=True)
    masked2 = jnp.where(col == i1, neg, masked)
    m2 = jnp.max(masked2, axis=1, keepdims=True)
    i2 = jnp.min(jnp.where(masked2 == m2, col, EPAD), axis=1, keepdims=True)
    ti_ref[...] = jnp.concatenate([i1, i2], axis=1)
    w1r = jax.nn.sigmoid(m1 - m2)
    w2r = jax.nn.sigmoid(m2 - m1)
    tw_ref[...] = jnp.concatenate([w1r, w2r], axis=1)


def _moe_ffn_kernel(be_ref, tok_ref, h_ref, w1_ref, w2_ref, wgt_ref,
                    out_ref, xg_ref, y_ref):
    b = pl.program_id(0)
    be = be_ref[b]

    @pl.when(b == 0)
    def _init():
        out_ref[...] = h_ref[...]

    @pl.when(be >= 0)
    def _work():
        base = b * BLK

        def gather(j, _):
            t = tok_ref[base + j]
            xg_ref[pl.ds(j, 1), :] = h_ref[pl.ds(t, 1), :]
            return 0

        jax.lax.fori_loop(0, BLK, gather, 0, unroll=8)
        a = jax.lax.dot_general(xg_ref[...], w1_ref[0], (((1,), (0,)), ((), ())),
                                preferred_element_type=jnp.float32)
        act = a * jax.nn.sigmoid(a)
        y = jax.lax.dot_general(act, w2_ref[0],
                                (((1,), (0,)), ((), ())),
                                preferred_element_type=jnp.float32)
        y_ref[...] = y * wgt_ref[0]

        def scatter(j, _):
            t = tok_ref[base + j]
            out_ref[pl.ds(t, 1), :] += y_ref[pl.ds(j, 1), :]
            return 0

        jax.lax.fori_loop(0, BLK, scatter, 0, unroll=8)


@functools.partial(jax.jit, static_argnums=())
def kernel(hidden_states, attn_W, attn_b, router_W, w1, w2):
    x = hidden_states.reshape(N, D)
    noise = (jax.random.normal(jax.random.key(1), hidden_states.shape,
                               hidden_states.dtype) * 0.0001).reshape(N, D)
    rw_pad = jnp.zeros((D, EPAD), jnp.float32).at[:, :E].set(router_W)

    h, ti, tw = pl.pallas_call(
        _attn_router_kernel,
        grid=(N // BN,),
        in_specs=[
            pl.BlockSpec((BN, D), lambda i: (i, 0)),
            pl.BlockSpec((BN, D), lambda i: (i, 0)),
            pl.BlockSpec((D, D), lambda i: (0, 0)),
            pl.BlockSpec((1, D), lambda i: (0, 0)),
            pl.BlockSpec((D, EPAD), lambda i: (0, 0)),
        ],
        out_specs=[
            pl.BlockSpec((BN, D), lambda i: (i, 0)),
            pl.BlockSpec((BN, TOPK), lambda i: (i, 0)),
            pl.BlockSpec((BN, TOPK), lambda i: (i, 0)),
        ],
        out_shape=[
            jax.ShapeDtypeStruct((N, D), jnp.float32),
            jax.ShapeDtypeStruct((N, TOPK), jnp.int32),
            jax.ShapeDtypeStruct((N, TOPK), jnp.float32),
        ],
    )(x, noise, attn_W, attn_b.reshape(1, D), rw_pad)

    ef = ti.reshape(-1)
    onehot = (ef[:, None] == jnp.arange(E, dtype=jnp.int32)[None, :]).astype(jnp.int32)
    csum = jnp.cumsum(onehot, axis=0)
    rank = jnp.take_along_axis(csum, ef[:, None], axis=1)[:, 0] - 1
    counts = csum[-1]
    padded = ((counts + BLK - 1) // BLK) * BLK
    ends = jnp.cumsum(padded)
    offs = ends - padded
    slot = offs[ef] + rank
    tok_of_slot = jnp.zeros((MAXS,), jnp.int32).at[slot].set(
        jnp.arange(N * TOPK, dtype=jnp.int32) // TOPK)
    wgt_of_slot = jnp.zeros((MAXS,), jnp.float32).at[slot].set(tw.reshape(-1))
    total = ends[-1]
    bstart = jnp.arange(NBLK, dtype=jnp.int32) * BLK
    be = jnp.searchsorted(ends, bstart, side='right').astype(jnp.int32)
    block_expert = jnp.where(bstart < total, be, -1)

    wgt3 = wgt_of_slot.reshape(NBLK, BLK, 1)

    out = pl.pallas_call(
        _moe_ffn_kernel,
        grid_spec=pltpu.PrefetchScalarGridSpec(
            num_scalar_prefetch=2,
            grid=(NBLK,),
            in_specs=[
                pl.BlockSpec((N, D), lambda b, be_r, tok_r: (0, 0)),
                pl.BlockSpec((1, D, FF),
                             lambda b, be_r, tok_r: (jnp.maximum(be_r[b], 0), 0, 0)),
                pl.BlockSpec((1, FF, D),
                             lambda b, be_r, tok_r: (jnp.maximum(be_r[b], 0), 0, 0)),
                pl.BlockSpec((1, BLK, 1), lambda b, be_r, tok_r: (b, 0, 0)),
            ],
            out_specs=pl.BlockSpec((N, D), lambda b, be_r, tok_r: (0, 0)),
            scratch_shapes=[
                pltpu.VMEM((BLK, D), jnp.float32),
                pltpu.VMEM((BLK, D), jnp.float32),
            ],
        ),
        out_shape=jax.ShapeDtypeStruct((N, D), jnp.float32),
    )(block_expert, tok_of_slot, h, w1, w2, wgt3)

    return out.reshape(hidden_states.shape)
